# Initial kernel scaffold; baseline (speedup 1.0000x reference)
import jax
import jax.numpy as jnp
from jax.experimental import pallas as pl


def kernel(outputs, targets):
    raise NotImplementedError("write your pallas kernel here")


# speedup vs baseline: 33.0960x; 33.0960x over previous
import functools

import jax
import jax.numpy as jnp
from jax import lax
from jax.experimental import pallas as pl
from jax.experimental.pallas import tpu as pltpu
from jax.experimental.pallas import tpu_sc as plsc

NIMG = 16
NPIX = 512 * 512
HALF = NPIX // 2
NB = 8192
EMAX = 8.0
SCALE = NB / EMAX
HSIZE = 3 * NB + 16
CH = 8192
NCH = HALF // CH
L = 16


def _body(x_hbm, t_hbm, out_hbm, xbuf, tbuf, hist, part, outrow, shared):
    cid = lax.axis_index("c")
    sid = lax.axis_index("s")
    img = cid * 8 + sid // 2
    half = sid % 2

    zeros16 = jnp.zeros((L,), jnp.float32)
    ones16 = jnp.ones((L,), jnp.float32)

    def zbody(i, carry):
        hist[pl.ds(i * L, L)] = zeros16
        return carry

    lax.fori_loop(0, HSIZE // L, zbody, 0)

    def acc_body(j, gvec):
        x = xbuf[pl.ds(j * L, L)]
        tf = tbuf[pl.ds(j * L, L)].astype(jnp.float32)
        e = 1.0 - x * (2.0 * tf - 1.0)
        m = e > 0.0
        ec = jnp.clip(e, 0.0, EMAX)
        bi = jnp.clip((ec * SCALE).astype(jnp.int32), 0, NB - 1)
        br = (NB - 1) - bi
        plsc.addupdate_scatter(hist, [br], ones16, mask=m)
        plsc.addupdate_scatter(hist, [br + NB], tf, mask=m)
        plsc.addupdate_scatter(hist, [br + 2 * NB], e, mask=m)
        return gvec + tf

    base = half * HALF
    gvec = zeros16
    for c in range(NCH):
        start = base + c * CH
        pltpu.sync_copy(x_hbm.at[img, pl.ds(start, CH)], xbuf)
        pltpu.sync_copy(t_hbm.at[img, pl.ds(start, CH)], tbuf)
        gvec = lax.fori_loop(0, CH // L, acc_body, gvec)
    hist[pl.ds(3 * NB, L)] = gvec

    pltpu.sync_copy(hist, shared.at[sid])
    plsc.subcore_barrier()

    @pl.when(half == 0)
    def _scan():
        pltpu.sync_copy(shared.at[sid + 1], part)
        gm = hist[pl.ds(3 * NB, L)] + part[pl.ds(3 * NB, L)]
        G = jnp.sum(gm)
        Gn = jnp.float32(NPIX) - G

        def sbody(k, carry):
            Rc, Pc, accv = carry
            n = hist[pl.ds(k * L, L)] + part[pl.ds(k * L, L)]
            p = hist[pl.ds(NB + k * L, L)] + part[pl.ds(NB + k * L, L)]
            s = hist[pl.ds(2 * NB + k * L, L)] + part[pl.ds(2 * NB + k * L, L)]
            cn = plsc.cumsum(n)
            cp = plsc.cumsum(p)
            Rend = Rc + cn
            Pend = Pc + cp
            Rstart = Rend - n
            Pstart = Pend - p
            jpe = (G - Pend) / jnp.maximum(G + Rend - Pend, 1.0)
            jps = (G - Pstart) / jnp.maximum(G + Rstart - Pstart, 1.0)
            jne = (Gn - (Rend - Pend)) / jnp.maximum(Gn + Pend, 1.0)
            jns = (Gn - (Rstart - Pstart)) / jnp.maximum(Gn + Pstart, 1.0)
            dj = (jps - jpe) + (jns - jne)
            emean = s / jnp.maximum(n, 1.0)
            contr = jnp.where(n > 0.0, emean * dj * 0.5, 0.0)
            return (Rc + jnp.sum(n), Pc + jnp.sum(p), accv + contr)

        init = (jnp.float32(0.0), jnp.float32(0.0), zeros16)
        _, _, accv = lax.fori_loop(0, NB // L, sbody, init)
        loss = jnp.sum(accv)
        outrow[...] = zeros16 + loss
        pltpu.sync_copy(outrow, out_hbm.at[img])


def _make_kernel():
    mesh = plsc.VectorSubcoreMesh(
        core_axis_name="c", subcore_axis_name="s", num_cores=2, num_subcores=16
    )

    return pl.kernel(
        _body,
        out_type=jax.ShapeDtypeStruct((NIMG, L), jnp.float32),
        mesh=mesh,
        compiler_params=pltpu.CompilerParams(needs_layout_passes=False),
        scratch_types=[
            pltpu.VMEM((CH,), jnp.float32),
            pltpu.VMEM((CH,), jnp.int32),
            pltpu.VMEM((HSIZE,), jnp.float32),
            pltpu.VMEM((HSIZE,), jnp.float32),
            pltpu.VMEM((L,), jnp.float32),
            pltpu.VMEM_SHARED((16, HSIZE), jnp.float32),
        ],
    )


@jax.jit
def kernel(outputs, targets):
    x = outputs.reshape(NIMG, NPIX)
    t = targets.astype(jnp.int32).reshape(NIMG, NPIX)
    per_image = _make_kernel()(x, t)
    return jnp.mean(per_image[:, 0])


# speedup vs baseline: 47.4210x; 1.4328x over previous
import functools

import jax
import jax.numpy as jnp
from jax import lax
from jax.experimental import pallas as pl
from jax.experimental.pallas import tpu as pltpu
from jax.experimental.pallas import tpu_sc as plsc

NIMG = 16
ROWS = 512
COLS = 512
NPIX = ROWS * COLS
NB = 4096
NBSHIFT = 12
EMAX = 8.0
SCALE = NB / EMAX
HSIZE = 3 * NB + 16
CROWS = 16
CH = CROWS * COLS
TROWS = ROWS // 2
NCH = TROWS // CROWS
VPR = COLS // 16
L = 16


def _body(x_hbm, t_hbm, out_hbm, xb0, tb0, xb1, tb1, hist, part, outrow,
          shared, sx0, st0, sx1, st1):
    cid = lax.axis_index("c")
    sid = lax.axis_index("s")
    img = cid * 8 + sid // 2
    half = sid % 2

    zeros16 = jnp.zeros((L,), jnp.float32)
    ones16 = jnp.ones((L,), jnp.float32)

    row0 = half * TROWS

    def start_chunk(c, xb, tb, sx, st):
        rs = row0 + jnp.minimum(c, NCH - 1) * CROWS
        cx = pltpu.make_async_copy(x_hbm.at[img, pl.ds(rs, CROWS), :], xb, sx)
        ct = pltpu.make_async_copy(t_hbm.at[img, pl.ds(rs, CROWS), :], tb, st)
        cx.start()
        ct.start()
        return cx, ct

    cx0, ct0 = start_chunk(jnp.int32(0), xb0, tb0, sx0, st0)

    def zbody(i, carry):
        hist[pl.ds(i * L, L)] = zeros16
        return carry

    lax.fori_loop(0, HSIZE // L, zbody, 0)

    def process(xb, tb, gvec):
        def row_body(r, g):
            for v in range(VPR):
                x = xb[r, pl.ds(v * L, L)]
                ti = tb[r, pl.ds(v * L, L)]
                tf = ti.astype(jnp.float32)
                e = 1.0 - x * (2.0 * tf - 1.0)
                m = e > 0.0
                ec = jnp.clip(e, 0.0, EMAX)
                bi = jnp.minimum((ec * SCALE).astype(jnp.int32), NB - 1)
                br = (NB - 1) - bi
                plsc.addupdate_scatter(hist, [br + ti * NB], ones16, mask=m)
                plsc.addupdate_scatter(hist, [br + 2 * NB], e, mask=m)
                g = g + tf
            return g
        return lax.fori_loop(0, CROWS, row_body, gvec)

    def pair_body(c2, gvec):
        c = 2 * c2
        cx1, ct1 = start_chunk(c + 1, xb1, tb1, sx1, st1)
        pltpu.make_async_copy(x_hbm.at[img, pl.ds(0, CROWS), :], xb0, sx0).wait()
        pltpu.make_async_copy(t_hbm.at[img, pl.ds(0, CROWS), :], tb0, st0).wait()
        gvec = process(xb0, tb0, gvec)
        start_chunk(c + 2, xb0, tb0, sx0, st0)
        cx1.wait()
        ct1.wait()
        gvec = process(xb1, tb1, gvec)
        return gvec

    gvec = lax.fori_loop(0, NCH // 2, pair_body, zeros16)
    pltpu.make_async_copy(x_hbm.at[img, pl.ds(0, CROWS), :], xb0, sx0).wait()
    pltpu.make_async_copy(t_hbm.at[img, pl.ds(0, CROWS), :], tb0, st0).wait()

    hist[pl.ds(3 * NB, L)] = gvec

    pltpu.sync_copy(hist, shared.at[sid])
    plsc.subcore_barrier()

    @pl.when(half == 0)
    def _scan():
        pltpu.sync_copy(shared.at[sid + 1], part)
        gm = hist[pl.ds(3 * NB, L)] + part[pl.ds(3 * NB, L)]
        G = jnp.sum(gm)
        Gn = jnp.float32(NPIX) - G

        def sbody(k, carry):
            Rc, Pc, accv = carry
            neg = hist[pl.ds(k * L, L)] + part[pl.ds(k * L, L)]
            pos = hist[pl.ds(NB + k * L, L)] + part[pl.ds(NB + k * L, L)]
            s = hist[pl.ds(2 * NB + k * L, L)] + part[pl.ds(2 * NB + k * L, L)]
            n = neg + pos
            cn = plsc.cumsum(n)
            cp = plsc.cumsum(pos)
            Rend = Rc + cn
            Pend = Pc + cp
            Rstart = Rend - n
            Pstart = Pend - pos
            jpe = (G - Pend) / jnp.maximum(G + Rend - Pend, 1.0)
            jps = (G - Pstart) / jnp.maximum(G + Rstart - Pstart, 1.0)
            jne = (Gn - (Rend - Pend)) / jnp.maximum(Gn + Pend, 1.0)
            jns = (Gn - (Rstart - Pstart)) / jnp.maximum(Gn + Pstart, 1.0)
            dj = (jps - jpe) + (jns - jne)
            emean = s / jnp.maximum(n, 1.0)
            contr = jnp.where(n > 0.0, emean * dj * 0.5, 0.0)
            return (Rc + jnp.sum(n), Pc + jnp.sum(pos), accv + contr)

        init = (jnp.float32(0.0), jnp.float32(0.0), zeros16)
        _, _, accv = lax.fori_loop(0, NB // L, sbody, init)
        loss = jnp.sum(accv)
        outrow[...] = zeros16 + loss
        pltpu.sync_copy(outrow, out_hbm.at[img])


def _make_kernel():
    mesh = plsc.VectorSubcoreMesh(
        core_axis_name="c", subcore_axis_name="s", num_cores=2, num_subcores=16
    )

    return pl.kernel(
        _body,
        out_type=jax.ShapeDtypeStruct((NIMG, L), jnp.float32),
        mesh=mesh,
        compiler_params=pltpu.CompilerParams(needs_layout_passes=False),
        scratch_types=[
            pltpu.VMEM((CROWS, COLS), jnp.float32),
            pltpu.VMEM((CROWS, COLS), jnp.int32),
            pltpu.VMEM((CROWS, COLS), jnp.float32),
            pltpu.VMEM((CROWS, COLS), jnp.int32),
            pltpu.VMEM((HSIZE,), jnp.float32),
            pltpu.VMEM((HSIZE,), jnp.float32),
            pltpu.VMEM((L,), jnp.float32),
            pltpu.VMEM_SHARED((16, HSIZE), jnp.float32),
            pltpu.SemaphoreType.DMA,
            pltpu.SemaphoreType.DMA,
            pltpu.SemaphoreType.DMA,
            pltpu.SemaphoreType.DMA,
        ],
    )


@jax.jit
def kernel(outputs, targets):
    t = targets.astype(jnp.int32)
    per_image = _make_kernel()(outputs, t)
    return jnp.mean(per_image[:, 0])


# speedup vs baseline: 163.9520x; 3.4574x over previous
import functools

import jax
import jax.numpy as jnp
from jax import lax
from jax.experimental import pallas as pl
from jax.experimental.pallas import tpu as pltpu
from jax.experimental.pallas import tpu_sc as plsc

NIMG = 16
ROWS = 512
COLS = 512
NPIX = ROWS * COLS
NB = 4096
EMAX = 8.0
SCALE = NB / EMAX
HSIZE = 2 * NB + 16
CROWS = 16
TROWS = ROWS // 2
NCH = TROWS // CROWS
VPR = COLS // 16
K = 8
L = 16


def _body(x_hbm, t_hbm, out_hbm, xb0, tb0, xb1, tb1, hist, part, outrow,
          shared, sx0, st0, sx1, st1):
    cid = lax.axis_index("c")
    sid = lax.axis_index("s")
    img = cid * 8 + sid // 2
    half = sid % 2

    zeros16 = jnp.zeros((L,), jnp.float32)
    ones16 = jnp.ones((L,), jnp.float32)

    row0 = half * TROWS

    def start_chunk(c, xb, tb, sx, st):
        rs = row0 + jnp.minimum(c, NCH - 1) * CROWS
        pltpu.make_async_copy(x_hbm.at[img, pl.ds(rs, CROWS), :], xb, sx).start()
        pltpu.make_async_copy(t_hbm.at[img, pl.ds(rs, CROWS), :], tb, st).start()

    def wait_chunk(xb, tb, sx, st):
        pltpu.make_async_copy(x_hbm.at[img, pl.ds(0, CROWS), :], xb, sx).wait()
        pltpu.make_async_copy(t_hbm.at[img, pl.ds(0, CROWS), :], tb, st).wait()

    start_chunk(jnp.int32(0), xb0, tb0, sx0, st0)

    def zbody(i, carry):
        hist[pl.ds(i * L, L)] = zeros16
        return carry

    lax.fori_loop(0, HSIZE // L, zbody, 0)

    def process(xb, tb, gvec):
        def load(v, r):
            return xb[r, pl.ds(v * L, L)], tb[r, pl.ds(v * L, L)]

        def compute(x, ti):
            pos = ti > 0
            e = jnp.where(pos, 1.0 - x, 1.0 + x)
            m = e > 0.0
            bf = jnp.minimum(e * SCALE, NB - 1.0)
            bi = bf.astype(jnp.int32)
            plane = ((NB - 1) - bi) + (ti << 12)
            return plane, m

        def scatter(items):
            for plane, m in items:
                plsc.addupdate_scatter(hist, [plane], ones16, mask=m)

        def tsum(tis):
            while len(tis) > 1:
                tis = [a + b for a, b in zip(tis[::2], tis[1::2])]
            return tis[0]

        def row_body(r, g):
            prev = None
            for gi in range(VPR // K):
                lv = [load(gi * K + v, r) for v in range(K)]
                cv = [compute(x, ti) for x, ti in lv]
                g = g + tsum([ti for _, ti in lv])
                if prev is not None:
                    scatter(prev)
                prev = cv
            scatter(prev)
            return g

        return lax.fori_loop(0, CROWS, row_body, gvec)

    def pair_body(c2, gvec):
        c = 2 * c2
        start_chunk(c + 1, xb1, tb1, sx1, st1)
        wait_chunk(xb0, tb0, sx0, st0)
        gvec = process(xb0, tb0, gvec)
        start_chunk(c + 2, xb0, tb0, sx0, st0)
        wait_chunk(xb1, tb1, sx1, st1)
        gvec = process(xb1, tb1, gvec)
        return gvec

    gvec = lax.fori_loop(0, NCH // 2, pair_body, jnp.zeros((L,), jnp.int32))
    wait_chunk(xb0, tb0, sx0, st0)

    hist[pl.ds(2 * NB, L)] = gvec.astype(jnp.float32)

    pltpu.sync_copy(hist, shared.at[sid])
    plsc.subcore_barrier()

    @pl.when(half == 0)
    def _scan():
        pltpu.sync_copy(shared.at[sid + 1], part)
        gm = hist[pl.ds(2 * NB, L)] + part[pl.ds(2 * NB, L)]
        G = jnp.sum(gm)
        Gn = jnp.float32(NPIX) - G
        emid0 = (jnp.float32(NB) - 0.5
                 - lax.iota(jnp.int32, L).astype(jnp.float32)) / SCALE
        estep = jnp.float32(L) / SCALE

        def sbody(k, carry):
            Rc, Pc, emid, accv = carry
            neg = hist[pl.ds(k * L, L)] + part[pl.ds(k * L, L)]
            pos = hist[pl.ds(NB + k * L, L)] + part[pl.ds(NB + k * L, L)]
            n = neg + pos
            cn = plsc.cumsum(n)
            cp = plsc.cumsum(pos)
            Rend = Rc + cn
            Pend = Pc + cp
            Rstart = Rend - n
            Pstart = Pend - pos
            jpe = (G - Pend) / jnp.maximum(G + Rend - Pend, 1.0)
            jps = (G - Pstart) / jnp.maximum(G + Rstart - Pstart, 1.0)
            jne = (Gn - (Rend - Pend)) / jnp.maximum(Gn + Pend, 1.0)
            jns = (Gn - (Rstart - Pstart)) / jnp.maximum(Gn + Pstart, 1.0)
            dj = (jps - jpe) + (jns - jne)
            return (Rc + jnp.sum(n), Pc + jnp.sum(pos), emid - estep,
                    accv + emid * dj)

        init = (jnp.float32(0.0), jnp.float32(0.0), emid0, zeros16)
        _, _, _, accv = lax.fori_loop(0, NB // L, sbody, init)
        loss = 0.5 * jnp.sum(accv)
        outrow[...] = zeros16 + loss
        pltpu.sync_copy(outrow, out_hbm.at[img])


def _make_kernel():
    mesh = plsc.VectorSubcoreMesh(
        core_axis_name="c", subcore_axis_name="s", num_cores=2, num_subcores=16
    )

    return pl.kernel(
        _body,
        out_type=jax.ShapeDtypeStruct((NIMG, L), jnp.float32),
        mesh=mesh,
        compiler_params=pltpu.CompilerParams(needs_layout_passes=False),
        scratch_types=[
            pltpu.VMEM((CROWS, COLS), jnp.float32),
            pltpu.VMEM((CROWS, COLS), jnp.int32),
            pltpu.VMEM((CROWS, COLS), jnp.float32),
            pltpu.VMEM((CROWS, COLS), jnp.int32),
            pltpu.VMEM((HSIZE,), jnp.float32),
            pltpu.VMEM((HSIZE,), jnp.float32),
            pltpu.VMEM((L,), jnp.float32),
            pltpu.VMEM_SHARED((16, HSIZE), jnp.float32),
            pltpu.SemaphoreType.DMA,
            pltpu.SemaphoreType.DMA,
            pltpu.SemaphoreType.DMA,
            pltpu.SemaphoreType.DMA,
        ],
    )


@jax.jit
def kernel(outputs, targets):
    t = targets.astype(jnp.int32)
    per_image = _make_kernel()(outputs, t)
    return jnp.mean(per_image[:, 0])
